# Initial kernel scaffold; baseline (speedup 1.0000x reference)
import jax
import jax.numpy as jnp
from jax.experimental import pallas as pl


def kernel(x, context_mask, speakers, W_ih_f, W_hh_f, b_ih_f, b_hh_f, W_ih_b, W_hh_b, b_ih_b, b_hh_b):
    raise NotImplementedError("write your pallas kernel here")


# speedup vs baseline: 21.9780x; 21.9780x over previous
import functools

import jax
import jax.numpy as jnp
from jax import lax
from jax.experimental import pallas as pl
from jax.experimental.pallas import tpu as pltpu

DIMS = 256
HIDS = 128
GATES = 4 * HIDS
NCL = 16
MTOK = 8 * 512
TRASH = MTOK * NCL
RPAD = MTOK * NCL + NCL
CH = 64
CHR = CH * NCL


def _gates_kernel(x_ref, w_ref, b_ref, o_ref):
    o_ref[...] = (
        jax.lax.dot_general(
            x_ref[...], w_ref[...], (((1,), (0,)), ((), ())),
            preferred_element_type=jnp.float32,
            precision=jax.lax.Precision.HIGHEST,
        )
        + b_ref[...]
    )


def _compute_gates(x_flat, wcat, bcat):
    return pl.pallas_call(
        _gates_kernel,
        grid=(8,),
        in_specs=[
            pl.BlockSpec((512, DIMS), lambda i: (i, 0)),
            pl.BlockSpec((DIMS, 2 * GATES), lambda i: (0, 0)),
            pl.BlockSpec((1, 2 * GATES), lambda i: (0, 0)),
        ],
        out_specs=pl.BlockSpec((512, 2 * GATES), lambda i: (i, 0)),
        out_shape=jax.ShapeDtypeStruct((MTOK, 2 * GATES), jnp.float32),
    )(x_flat, wcat, bcat)


def _cell(g, h, c, w_ref):
    gates = g + jax.lax.dot_general(
        h, w_ref[...], (((1,), (0,)), ((), ())),
        preferred_element_type=jnp.float32,
        precision=jax.lax.Precision.HIGHEST,
    )
    gi = jax.nn.sigmoid(gates[:, 0:HIDS])
    gf = jax.nn.sigmoid(gates[:, HIDS:2 * HIDS])
    gg = jnp.tanh(gates[:, 2 * HIDS:3 * HIDS])
    go = jax.nn.sigmoid(gates[:, 3 * HIDS:4 * HIDS])
    c_new = gf * c + gi * gg
    h_new = go * jnp.tanh(c_new)
    return h_new, c_new


def _rec_kernel(df_ref, whf_ref, whb_ref, pgf_hbm, pgb_hbm, hf_hbm, hb_hbm,
                buf_f, buf_b, obuf_f, obuf_b, sif, sib, sof, sob):
    dv = jnp.where(df_ref[...] == TRASH, -1, df_ref[...])
    maxv = jnp.max(dv)
    max_len = (maxv + NCL) // NCL
    nchunks = (max_len + CH - 1) // CH

    obuf_f[0, 0:NCL, :] = jnp.zeros((NCL, HIDS), jnp.float32)
    obuf_b[0, 0:NCL, :] = jnp.zeros((NCL, HIDS), jnp.float32)
    zf = pltpu.make_async_copy(obuf_f.at[0, pl.ds(0, NCL)],
                               hf_hbm.at[pl.ds(TRASH, NCL)], sof.at[0])
    zb = pltpu.make_async_copy(obuf_b.at[0, pl.ds(0, NCL)],
                               hb_hbm.at[pl.ds(TRASH, NCL)], sob.at[0])
    zf.start()
    zb.start()
    zf.wait()
    zb.wait()

    def start_in(c, slot):
        pltpu.make_async_copy(pgf_hbm.at[pl.ds(c * CHR, CHR)],
                              buf_f.at[slot], sif.at[slot]).start()
        pltpu.make_async_copy(pgb_hbm.at[pl.ds(c * CHR, CHR)],
                              buf_b.at[slot], sib.at[slot]).start()

    @pl.when(nchunks > 0)
    def _():
        start_in(0, 0)

    def chunk_body(c, carry):
        h_f, c_f, h_b, c_b = carry
        slot = lax.rem(c, 2)
        pltpu.make_async_copy(pgf_hbm.at[pl.ds(c * CHR, CHR)],
                              buf_f.at[slot], sif.at[slot]).wait()
        pltpu.make_async_copy(pgb_hbm.at[pl.ds(c * CHR, CHR)],
                              buf_b.at[slot], sib.at[slot]).wait()

        @pl.when(c + 1 < nchunks)
        def _():
            start_in(c + 1, lax.rem(c + 1, 2))

        @pl.when(c >= 2)
        def _():
            pltpu.make_async_copy(obuf_f.at[slot],
                                  hf_hbm.at[pl.ds((c - 2) * CHR, CHR)],
                                  sof.at[slot]).wait()
            pltpu.make_async_copy(obuf_b.at[slot],
                                  hb_hbm.at[pl.ds((c - 2) * CHR, CHR)],
                                  sob.at[slot]).wait()

        def step(t, sc):
            h_f, c_f, h_b, c_b = sc
            base = t * NCL
            gf = buf_f[slot, pl.ds(base, NCL), :]
            gb = buf_b[slot, pl.ds(base, NCL), :]
            h_f, c_f = _cell(gf, h_f, c_f, whf_ref)
            h_b, c_b = _cell(gb, h_b, c_b, whb_ref)
            obuf_f[slot, pl.ds(base, NCL), :] = h_f
            obuf_b[slot, pl.ds(base, NCL), :] = h_b
            return h_f, c_f, h_b, c_b

        h_f, c_f, h_b, c_b = lax.fori_loop(0, CH, step, (h_f, c_f, h_b, c_b))

        pltpu.make_async_copy(obuf_f.at[slot],
                              hf_hbm.at[pl.ds(c * CHR, CHR)],
                              sof.at[slot]).start()
        pltpu.make_async_copy(obuf_b.at[slot],
                              hb_hbm.at[pl.ds(c * CHR, CHR)],
                              sob.at[slot]).start()
        return h_f, c_f, h_b, c_b

    z = jnp.zeros((NCL, HIDS), jnp.float32)
    lax.fori_loop(0, nchunks, chunk_body, (z, z, z, z))

    @pl.when(nchunks >= 2)
    def _():
        c = nchunks - 2
        slot = lax.rem(c, 2)
        pltpu.make_async_copy(obuf_f.at[slot],
                              hf_hbm.at[pl.ds(c * CHR, CHR)], sof.at[slot]).wait()
        pltpu.make_async_copy(obuf_b.at[slot],
                              hb_hbm.at[pl.ds(c * CHR, CHR)], sob.at[slot]).wait()

    @pl.when(nchunks >= 1)
    def _():
        c = nchunks - 1
        slot = lax.rem(c, 2)
        pltpu.make_async_copy(obuf_f.at[slot],
                              hf_hbm.at[pl.ds(c * CHR, CHR)], sof.at[slot]).wait()
        pltpu.make_async_copy(obuf_b.at[slot],
                              hb_hbm.at[pl.ds(c * CHR, CHR)], sob.at[slot]).wait()


def _run_recurrence(d_f2d, whf_t, whb_t, pg_f, pg_b):
    return pl.pallas_call(
        _rec_kernel,
        in_specs=[
            pl.BlockSpec(memory_space=pltpu.MemorySpace.VMEM),
            pl.BlockSpec(memory_space=pltpu.MemorySpace.VMEM),
            pl.BlockSpec(memory_space=pltpu.MemorySpace.VMEM),
            pl.BlockSpec(memory_space=pltpu.MemorySpace.HBM),
            pl.BlockSpec(memory_space=pltpu.MemorySpace.HBM),
        ],
        out_specs=[
            pl.BlockSpec(memory_space=pltpu.MemorySpace.HBM),
            pl.BlockSpec(memory_space=pltpu.MemorySpace.HBM),
        ],
        out_shape=[
            jax.ShapeDtypeStruct((RPAD, HIDS), jnp.float32),
            jax.ShapeDtypeStruct((RPAD, HIDS), jnp.float32),
        ],
        scratch_shapes=[
            pltpu.VMEM((2, CHR, GATES), jnp.float32),
            pltpu.VMEM((2, CHR, GATES), jnp.float32),
            pltpu.VMEM((2, CHR, HIDS), jnp.float32),
            pltpu.VMEM((2, CHR, HIDS), jnp.float32),
            pltpu.SemaphoreType.DMA((2,)),
            pltpu.SemaphoreType.DMA((2,)),
            pltpu.SemaphoreType.DMA((2,)),
            pltpu.SemaphoreType.DMA((2,)),
        ],
    )(d_f2d, whf_t, whb_t, pg_f, pg_b)


def kernel(x, context_mask, speakers, W_ih_f, W_hh_f, b_ih_f, b_hh_f,
           W_ih_b, W_hh_b, b_ih_b, b_hh_b):
    B, S, d = x.shape
    x_flat = x.reshape(MTOK, d)
    m = context_mask.reshape(MTOK)
    spk = speakers.reshape(MTOK).astype(jnp.int32)

    oh = (spk[:, None] == jnp.arange(NCL, dtype=jnp.int32)[None, :]) & m[:, None]
    csum = jnp.cumsum(oh.astype(jnp.int32), axis=0)
    col = jnp.take_along_axis(csum, spk[:, None], axis=1)[:, 0] - 1
    seqlens = csum[-1]
    d_f = jnp.where(m, col * NCL + spk, TRASH)
    lens = seqlens[spk]
    d_b = jnp.where(m, (lens - 1 - col) * NCL + spk, TRASH)

    wcat = jnp.concatenate([W_ih_f.T, W_ih_b.T], axis=1)
    bcat = jnp.concatenate([b_ih_f + b_hh_f, b_ih_b + b_hh_b]).reshape(1, 2 * GATES)
    gates = _compute_gates(x_flat, wcat, bcat)

    pg_f = jnp.zeros((RPAD, GATES), jnp.float32).at[d_f].set(gates[:, :GATES])
    pg_b = jnp.zeros((RPAD, GATES), jnp.float32).at[d_b].set(gates[:, GATES:])

    d_f2d = d_f.reshape(32, 128)
    hf_buf, hb_buf = _run_recurrence(d_f2d, W_hh_f.T, W_hh_b.T, pg_f, pg_b)

    out = jnp.concatenate([hf_buf[d_f], hb_buf[d_b]], axis=-1)
    return out.reshape(B, S, 2 * HIDS)


# speedup vs baseline: 73.7796x; 3.3570x over previous
import functools

import jax
import jax.numpy as jnp
from jax import lax
from jax.experimental import pallas as pl
from jax.experimental.pallas import tpu as pltpu
from jax.experimental.pallas import tpu_sc as plsc

DIMS = 256
HIDS = 128
GATES = 4 * HIDS
NCL = 16
MTOK = 8 * 512
TRASH = MTOK * NCL
RPAD = MTOK * NCL + NCL
CH = 64
CHR = CH * NCL


def _gates_kernel(x_ref, w_ref, b_ref, of_ref, ob_ref):
    g = (
        jax.lax.dot_general(
            x_ref[...], w_ref[...], (((1,), (0,)), ((), ())),
            preferred_element_type=jnp.float32,
            precision=jax.lax.Precision.HIGHEST,
        )
        + b_ref[...]
    )
    of_ref[...] = g[:, :GATES]
    ob_ref[...] = g[:, GATES:]


def _compute_gates(x_flat, wcat, bcat):
    return pl.pallas_call(
        _gates_kernel,
        grid=(8,),
        in_specs=[
            pl.BlockSpec((512, DIMS), lambda i: (i, 0)),
            pl.BlockSpec((DIMS, 2 * GATES), lambda i: (0, 0)),
            pl.BlockSpec((1, 2 * GATES), lambda i: (0, 0)),
        ],
        out_specs=[
            pl.BlockSpec((512, GATES), lambda i: (i, 0)),
            pl.BlockSpec((512, GATES), lambda i: (i, 0)),
        ],
        out_shape=[
            jax.ShapeDtypeStruct((MTOK, GATES), jnp.float32),
            jax.ShapeDtypeStruct((MTOK, GATES), jnp.float32),
        ],
    )(x_flat, wcat, bcat)


NSUB = 16
TPW = MTOK // 32
NCHK = MTOK // NSUB


def _sc_group_scatter_body(spk_hbm, mask_hbm, gf_hbm, gb_hbm,
                           destf_hbm, destb_hbm, pgf_hbm, pgb_hbm,
                           spk_v, mask_v, col_v, df_v, db_v, tmp_v,
                           outf_v, outb_v, rows_v, sh_f, sh_b, sem):
    c = lax.axis_index("c")
    s = lax.axis_index("s")
    k = s

    pltpu.sync_copy(spk_hbm, spk_v)
    pltpu.sync_copy(mask_hbm, mask_v)

    def p1(i, counter):
        sv = spk_v[pl.ds(i * NSUB, NSUB)]
        mv = mask_v[pl.ds(i * NSUB, NSUB)]
        m = (sv == k) & (mv != 0)
        mi = jnp.where(m, jnp.int32(1), jnp.int32(0))
        pc = plsc.cumsum(mi)
        col_v[pl.ds(i * NSUB, NSUB)] = jnp.where(m, counter + pc - 1, -1)
        return counter + jnp.sum(mi)

    len_k = lax.fori_loop(0, NCHK, p1, jnp.int32(0))

    def p2(i, carry):
        cv = col_v[pl.ds(i * NSUB, NSUB)]
        m = cv >= 0
        df_v[pl.ds(i * NSUB, NSUB)] = jnp.where(m, cv * NCL + k + 1, 0)
        db_v[pl.ds(i * NSUB, NSUB)] = jnp.where(
            m, (len_k - 1 - cv) * NCL + k + 1, 0)
        return carry

    lax.fori_loop(0, NCHK, p2, jnp.int32(0))

    pltpu.sync_copy(df_v, sh_f.at[s])
    pltpu.sync_copy(db_v, sh_b.at[s])
    plsc.subcore_barrier()

    wid = c * NSUB + s
    base = wid * TPW

    def merge(sh, out_v):
        pltpu.sync_copy(sh.at[:, pl.ds(base, TPW)], tmp_v)
        for j in range(TPW // NSUB):
            acc = jnp.zeros((NSUB,), jnp.int32)
            for r in range(NSUB):
                acc = acc + tmp_v[r, pl.ds(j * NSUB, NSUB)]
            out_v[pl.ds(j * NSUB, NSUB)] = jnp.where(acc > 0, acc - 1,
                                                     jnp.int32(TRASH))

    merge(sh_f, outf_v)
    merge(sh_b, outb_v)
    pltpu.sync_copy(outf_v, destf_hbm.at[pl.ds(base, TPW)])
    pltpu.sync_copy(outb_v, destb_hbm.at[pl.ds(base, TPW)])

    pltpu.sync_copy(gf_hbm.at[pl.ds(base, TPW)], rows_v)
    pltpu.async_copy(rows_v, pgf_hbm.at[outf_v], sem).wait()
    pltpu.sync_copy(gb_hbm.at[pl.ds(base, TPW)], rows_v)
    pltpu.async_copy(rows_v, pgb_hbm.at[outb_v], sem).wait()


def _sc_group_scatter(spk, mask_i, gates_f, gates_b):
    mesh = plsc.VectorSubcoreMesh(core_axis_name="c", subcore_axis_name="s")
    f = pl.kernel(
        _sc_group_scatter_body,
        out_type=[
            jax.ShapeDtypeStruct((MTOK,), jnp.int32),
            jax.ShapeDtypeStruct((MTOK,), jnp.int32),
            jax.ShapeDtypeStruct((RPAD, GATES), jnp.float32),
            jax.ShapeDtypeStruct((RPAD, GATES), jnp.float32),
        ],
        mesh=mesh,
        scratch_types=[
            pltpu.VMEM((MTOK,), jnp.int32),
            pltpu.VMEM((MTOK,), jnp.int32),
            pltpu.VMEM((MTOK,), jnp.int32),
            pltpu.VMEM((MTOK,), jnp.int32),
            pltpu.VMEM((MTOK,), jnp.int32),
            pltpu.VMEM((NSUB, TPW), jnp.int32),
            pltpu.VMEM((TPW,), jnp.int32),
            pltpu.VMEM((TPW,), jnp.int32),
            pltpu.VMEM((TPW, GATES), jnp.float32),
            pltpu.MemorySpace.VMEM_SHARED((NSUB, MTOK), jnp.int32),
            pltpu.MemorySpace.VMEM_SHARED((NSUB, MTOK), jnp.int32),
            pltpu.SemaphoreType.DMA,
        ],
        compiler_params=pltpu.CompilerParams(needs_layout_passes=False),
    )
    return f(spk, mask_i, gates_f, gates_b)


def _sc_gather_body(hf_hbm, hb_hbm, destf_hbm, destb_hbm, out_hbm,
                    idx_v, rowsf_v, rowsb_v, sem):
    c = lax.axis_index("c")
    s = lax.axis_index("s")
    wid = c * NSUB + s
    base = wid * TPW
    pltpu.sync_copy(destf_hbm.at[pl.ds(base, TPW)], idx_v)
    pltpu.async_copy(hf_hbm.at[idx_v], rowsf_v, sem).wait()
    pltpu.sync_copy(destb_hbm.at[pl.ds(base, TPW)], idx_v)
    pltpu.async_copy(hb_hbm.at[idx_v], rowsb_v, sem).wait()
    pltpu.sync_copy(rowsf_v, out_hbm.at[pl.ds(base, TPW), pl.ds(0, HIDS)])
    pltpu.sync_copy(rowsb_v, out_hbm.at[pl.ds(base, TPW), pl.ds(HIDS, HIDS)])


def _sc_gather(hf_buf, hb_buf, dest_f, dest_b):
    mesh = plsc.VectorSubcoreMesh(core_axis_name="c", subcore_axis_name="s")
    f = pl.kernel(
        _sc_gather_body,
        out_type=jax.ShapeDtypeStruct((MTOK, 2 * HIDS), jnp.float32),
        mesh=mesh,
        scratch_types=[
            pltpu.VMEM((TPW,), jnp.int32),
            pltpu.VMEM((TPW, HIDS), jnp.float32),
            pltpu.VMEM((TPW, HIDS), jnp.float32),
            pltpu.SemaphoreType.DMA,
        ],
        compiler_params=pltpu.CompilerParams(needs_layout_passes=False),
    )
    return f(hf_buf, hb_buf, dest_f, dest_b)


def _cell(g, h, c, w_ref):
    gates = g + jax.lax.dot_general(
        h, w_ref[...], (((1,), (0,)), ((), ())),
        preferred_element_type=jnp.float32,
        precision=jax.lax.Precision.HIGHEST,
    )
    gi = jax.nn.sigmoid(gates[:, 0:HIDS])
    gf = jax.nn.sigmoid(gates[:, HIDS:2 * HIDS])
    gg = jnp.tanh(gates[:, 2 * HIDS:3 * HIDS])
    go = jax.nn.sigmoid(gates[:, 3 * HIDS:4 * HIDS])
    c_new = gf * c + gi * gg
    h_new = go * jnp.tanh(c_new)
    return h_new, c_new


def _rec_kernel(df_ref, whf_ref, whb_ref, pgf_hbm, pgb_hbm, hf_hbm, hb_hbm,
                buf_f, buf_b, obuf_f, obuf_b, sif, sib, sof, sob):
    dv = jnp.where(df_ref[...] == TRASH, -1, df_ref[...])
    maxv = jnp.max(dv)
    max_len = (maxv + NCL) // NCL
    nchunks = (max_len + CH - 1) // CH

    obuf_f[0, 0:NCL, :] = jnp.zeros((NCL, HIDS), jnp.float32)
    obuf_b[0, 0:NCL, :] = jnp.zeros((NCL, HIDS), jnp.float32)
    zf = pltpu.make_async_copy(obuf_f.at[0, pl.ds(0, NCL)],
                               hf_hbm.at[pl.ds(TRASH, NCL)], sof.at[0])
    zb = pltpu.make_async_copy(obuf_b.at[0, pl.ds(0, NCL)],
                               hb_hbm.at[pl.ds(TRASH, NCL)], sob.at[0])
    zf.start()
    zb.start()
    zf.wait()
    zb.wait()

    def start_in(c, slot):
        pltpu.make_async_copy(pgf_hbm.at[pl.ds(c * CHR, CHR)],
                              buf_f.at[slot], sif.at[slot]).start()
        pltpu.make_async_copy(pgb_hbm.at[pl.ds(c * CHR, CHR)],
                              buf_b.at[slot], sib.at[slot]).start()

    @pl.when(nchunks > 0)
    def _():
        start_in(0, 0)

    def chunk_body(c, carry):
        h_f, c_f, h_b, c_b = carry
        slot = lax.rem(c, 2)
        pltpu.make_async_copy(pgf_hbm.at[pl.ds(c * CHR, CHR)],
                              buf_f.at[slot], sif.at[slot]).wait()
        pltpu.make_async_copy(pgb_hbm.at[pl.ds(c * CHR, CHR)],
                              buf_b.at[slot], sib.at[slot]).wait()

        @pl.when(c + 1 < nchunks)
        def _():
            start_in(c + 1, lax.rem(c + 1, 2))

        @pl.when(c >= 2)
        def _():
            pltpu.make_async_copy(obuf_f.at[slot],
                                  hf_hbm.at[pl.ds((c - 2) * CHR, CHR)],
                                  sof.at[slot]).wait()
            pltpu.make_async_copy(obuf_b.at[slot],
                                  hb_hbm.at[pl.ds((c - 2) * CHR, CHR)],
                                  sob.at[slot]).wait()

        def step(t, sc):
            h_f, c_f, h_b, c_b = sc
            base = t * NCL
            gf = buf_f[slot, pl.ds(base, NCL), :]
            gb = buf_b[slot, pl.ds(base, NCL), :]
            h_f, c_f = _cell(gf, h_f, c_f, whf_ref)
            h_b, c_b = _cell(gb, h_b, c_b, whb_ref)
            obuf_f[slot, pl.ds(base, NCL), :] = h_f
            obuf_b[slot, pl.ds(base, NCL), :] = h_b
            return h_f, c_f, h_b, c_b

        h_f, c_f, h_b, c_b = lax.fori_loop(0, CH, step, (h_f, c_f, h_b, c_b))

        pltpu.make_async_copy(obuf_f.at[slot],
                              hf_hbm.at[pl.ds(c * CHR, CHR)],
                              sof.at[slot]).start()
        pltpu.make_async_copy(obuf_b.at[slot],
                              hb_hbm.at[pl.ds(c * CHR, CHR)],
                              sob.at[slot]).start()
        return h_f, c_f, h_b, c_b

    z = jnp.zeros((NCL, HIDS), jnp.float32)
    lax.fori_loop(0, nchunks, chunk_body, (z, z, z, z))

    @pl.when(nchunks >= 2)
    def _():
        c = nchunks - 2
        slot = lax.rem(c, 2)
        pltpu.make_async_copy(obuf_f.at[slot],
                              hf_hbm.at[pl.ds(c * CHR, CHR)], sof.at[slot]).wait()
        pltpu.make_async_copy(obuf_b.at[slot],
                              hb_hbm.at[pl.ds(c * CHR, CHR)], sob.at[slot]).wait()

    @pl.when(nchunks >= 1)
    def _():
        c = nchunks - 1
        slot = lax.rem(c, 2)
        pltpu.make_async_copy(obuf_f.at[slot],
                              hf_hbm.at[pl.ds(c * CHR, CHR)], sof.at[slot]).wait()
        pltpu.make_async_copy(obuf_b.at[slot],
                              hb_hbm.at[pl.ds(c * CHR, CHR)], sob.at[slot]).wait()


def _run_recurrence(d_f2d, whf_t, whb_t, pg_f, pg_b):
    return pl.pallas_call(
        _rec_kernel,
        in_specs=[
            pl.BlockSpec(memory_space=pltpu.MemorySpace.VMEM),
            pl.BlockSpec(memory_space=pltpu.MemorySpace.VMEM),
            pl.BlockSpec(memory_space=pltpu.MemorySpace.VMEM),
            pl.BlockSpec(memory_space=pltpu.MemorySpace.HBM),
            pl.BlockSpec(memory_space=pltpu.MemorySpace.HBM),
        ],
        out_specs=[
            pl.BlockSpec(memory_space=pltpu.MemorySpace.HBM),
            pl.BlockSpec(memory_space=pltpu.MemorySpace.HBM),
        ],
        out_shape=[
            jax.ShapeDtypeStruct((RPAD, HIDS), jnp.float32),
            jax.ShapeDtypeStruct((RPAD, HIDS), jnp.float32),
        ],
        scratch_shapes=[
            pltpu.VMEM((2, CHR, GATES), jnp.float32),
            pltpu.VMEM((2, CHR, GATES), jnp.float32),
            pltpu.VMEM((2, CHR, HIDS), jnp.float32),
            pltpu.VMEM((2, CHR, HIDS), jnp.float32),
            pltpu.SemaphoreType.DMA((2,)),
            pltpu.SemaphoreType.DMA((2,)),
            pltpu.SemaphoreType.DMA((2,)),
            pltpu.SemaphoreType.DMA((2,)),
        ],
    )(d_f2d, whf_t, whb_t, pg_f, pg_b)


def kernel(x, context_mask, speakers, W_ih_f, W_hh_f, b_ih_f, b_hh_f,
           W_ih_b, W_hh_b, b_ih_b, b_hh_b):
    B, S, d = x.shape
    x_flat = x.reshape(MTOK, d)
    mask_i = context_mask.reshape(MTOK).astype(jnp.int32)
    spk = speakers.reshape(MTOK).astype(jnp.int32)

    wcat = jnp.concatenate([W_ih_f.T, W_ih_b.T], axis=1)
    bcat = jnp.concatenate([b_ih_f + b_hh_f, b_ih_b + b_hh_b]).reshape(1, 2 * GATES)
    gates_f, gates_b = _compute_gates(x_flat, wcat, bcat)

    dest_f, dest_b, pg_f, pg_b = _sc_group_scatter(spk, mask_i, gates_f, gates_b)

    d_f2d = dest_f.reshape(32, 128)
    hf_buf, hb_buf = _run_recurrence(d_f2d, W_hh_f.T, W_hh_b.T, pg_f, pg_b)

    out = _sc_gather(hf_buf, hb_buf, dest_f, dest_b)
    return out.reshape(B, S, 2 * HIDS)


# speedup vs baseline: 79.2312x; 1.0739x over previous
import functools

import jax
import jax.numpy as jnp
from jax import lax
from jax.experimental import pallas as pl
from jax.experimental.pallas import tpu as pltpu
from jax.experimental.pallas import tpu_sc as plsc

DIMS = 256
HIDS = 128
GATES = 4 * HIDS
NCL = 16
MTOK = 8 * 512
TRASH = MTOK * NCL
RPAD = MTOK * NCL + NCL
CH = 64
CHR = CH * NCL


def _gates_kernel(x_ref, w_ref, b_ref, of_ref, ob_ref):
    g = (
        jax.lax.dot_general(
            x_ref[...], w_ref[...], (((1,), (0,)), ((), ())),
            preferred_element_type=jnp.float32,
            precision=jax.lax.Precision.HIGHEST,
        )
        + b_ref[...]
    )
    of_ref[...] = g[:, :GATES]
    ob_ref[...] = g[:, GATES:]


def _compute_gates(x_flat, wcat, bcat):
    return pl.pallas_call(
        _gates_kernel,
        grid=(8,),
        in_specs=[
            pl.BlockSpec((512, DIMS), lambda i: (i, 0)),
            pl.BlockSpec((DIMS, 2 * GATES), lambda i: (0, 0)),
            pl.BlockSpec((1, 2 * GATES), lambda i: (0, 0)),
        ],
        out_specs=[
            pl.BlockSpec((512, GATES), lambda i: (i, 0)),
            pl.BlockSpec((512, GATES), lambda i: (i, 0)),
        ],
        out_shape=[
            jax.ShapeDtypeStruct((MTOK, GATES), jnp.float32),
            jax.ShapeDtypeStruct((MTOK, GATES), jnp.float32),
        ],
    )(x_flat, wcat, bcat)


NSUB = 16
TPW = MTOK // 32
NCHK = MTOK // NSUB


def _sc_group_scatter_body(spk_hbm, mask_hbm, gf_hbm, gb_hbm,
                           destf_hbm, destb_hbm, pgf_hbm, pgb_hbm,
                           spk_v, mask_v, col_v, df_v, db_v, tmp_v,
                           outf_v, outb_v, rows_v, sh_f, sh_b, sem):
    c = lax.axis_index("c")
    s = lax.axis_index("s")
    k = s

    pltpu.sync_copy(spk_hbm, spk_v)
    pltpu.sync_copy(mask_hbm, mask_v)

    def p1(i, counter):
        sv = spk_v[pl.ds(i * NSUB, NSUB)]
        mv = mask_v[pl.ds(i * NSUB, NSUB)]
        m = (sv == k) & (mv != 0)
        mi = jnp.where(m, jnp.int32(1), jnp.int32(0))
        pc = plsc.cumsum(mi)
        col_v[pl.ds(i * NSUB, NSUB)] = jnp.where(m, counter + pc - 1, -1)
        return counter + jnp.sum(mi)

    len_k = lax.fori_loop(0, NCHK, p1, jnp.int32(0))

    def p2(i, carry):
        cv = col_v[pl.ds(i * NSUB, NSUB)]
        m = cv >= 0
        df_v[pl.ds(i * NSUB, NSUB)] = jnp.where(m, cv * NCL + k + 1, 0)
        db_v[pl.ds(i * NSUB, NSUB)] = jnp.where(
            m, (len_k - 1 - cv) * NCL + k + 1, 0)
        return carry

    lax.fori_loop(0, NCHK, p2, jnp.int32(0))

    pltpu.sync_copy(df_v, sh_f.at[s])
    pltpu.sync_copy(db_v, sh_b.at[s])
    plsc.subcore_barrier()

    wid = c * NSUB + s
    base = wid * TPW

    def merge(sh, out_v):
        pltpu.sync_copy(sh.at[:, pl.ds(base, TPW)], tmp_v)
        for j in range(TPW // NSUB):
            acc = jnp.zeros((NSUB,), jnp.int32)
            for r in range(NSUB):
                acc = acc + tmp_v[r, pl.ds(j * NSUB, NSUB)]
            out_v[pl.ds(j * NSUB, NSUB)] = jnp.where(acc > 0, acc - 1,
                                                     jnp.int32(TRASH))

    merge(sh_f, outf_v)
    merge(sh_b, outb_v)
    pltpu.sync_copy(outf_v, destf_hbm.at[pl.ds(base, TPW)])
    pltpu.sync_copy(outb_v, destb_hbm.at[pl.ds(base, TPW)])

    pltpu.sync_copy(gf_hbm.at[pl.ds(base, TPW)], rows_v)
    pltpu.async_copy(rows_v, pgf_hbm.at[outf_v], sem).wait()
    pltpu.sync_copy(gb_hbm.at[pl.ds(base, TPW)], rows_v)
    pltpu.async_copy(rows_v, pgb_hbm.at[outb_v], sem).wait()


def _sc_group_scatter(spk, mask_i, gates_f, gates_b):
    mesh = plsc.VectorSubcoreMesh(core_axis_name="c", subcore_axis_name="s")
    f = pl.kernel(
        _sc_group_scatter_body,
        out_type=[
            jax.ShapeDtypeStruct((MTOK,), jnp.int32),
            jax.ShapeDtypeStruct((MTOK,), jnp.int32),
            jax.ShapeDtypeStruct((RPAD, GATES), jnp.float32),
            jax.ShapeDtypeStruct((RPAD, GATES), jnp.float32),
        ],
        mesh=mesh,
        scratch_types=[
            pltpu.VMEM((MTOK,), jnp.int32),
            pltpu.VMEM((MTOK,), jnp.int32),
            pltpu.VMEM((MTOK,), jnp.int32),
            pltpu.VMEM((MTOK,), jnp.int32),
            pltpu.VMEM((MTOK,), jnp.int32),
            pltpu.VMEM((NSUB, TPW), jnp.int32),
            pltpu.VMEM((TPW,), jnp.int32),
            pltpu.VMEM((TPW,), jnp.int32),
            pltpu.VMEM((TPW, GATES), jnp.float32),
            pltpu.MemorySpace.VMEM_SHARED((NSUB, MTOK), jnp.int32),
            pltpu.MemorySpace.VMEM_SHARED((NSUB, MTOK), jnp.int32),
            pltpu.SemaphoreType.DMA,
        ],
        compiler_params=pltpu.CompilerParams(needs_layout_passes=False),
    )
    return f(spk, mask_i, gates_f, gates_b)


def _sc_gather_body(h_hbm, destf_hbm, destb_hbm, out_hbm,
                    idx_v, rowsf_v, rowsb_v, sem):
    c = lax.axis_index("c")
    s = lax.axis_index("s")
    wid = c * NSUB + s
    base = wid * TPW
    pltpu.sync_copy(destf_hbm.at[pl.ds(base, TPW)], idx_v)
    pltpu.async_copy(h_hbm.at[idx_v], rowsf_v, sem).wait()
    pltpu.sync_copy(destb_hbm.at[pl.ds(base, TPW)], idx_v)
    pltpu.async_copy(h_hbm.at[idx_v], rowsb_v, sem).wait()
    pltpu.sync_copy(rowsf_v.at[:, pl.ds(0, HIDS)],
                    out_hbm.at[pl.ds(base, TPW), pl.ds(0, HIDS)])
    pltpu.sync_copy(rowsb_v.at[:, pl.ds(HIDS, HIDS)],
                    out_hbm.at[pl.ds(base, TPW), pl.ds(HIDS, HIDS)])


def _sc_gather(h_buf, dest_f, dest_b):
    mesh = plsc.VectorSubcoreMesh(core_axis_name="c", subcore_axis_name="s")
    f = pl.kernel(
        _sc_gather_body,
        out_type=jax.ShapeDtypeStruct((MTOK, 2 * HIDS), jnp.float32),
        mesh=mesh,
        scratch_types=[
            pltpu.VMEM((TPW,), jnp.int32),
            pltpu.VMEM((TPW, 2 * HIDS), jnp.float32),
            pltpu.VMEM((TPW, 2 * HIDS), jnp.float32),
            pltpu.SemaphoreType.DMA,
        ],
        compiler_params=pltpu.CompilerParams(needs_layout_passes=False),
    )
    return f(h_buf, dest_f, dest_b)


def _rec_kernel(df_ref, wbd_ref, pgf_hbm, pgb_hbm, h_hbm,
                buf, obuf, sif, sib, soh):
    dv = jnp.where(df_ref[...] == TRASH, -1, df_ref[...])
    maxv = jnp.max(dv)
    max_len = (maxv + NCL) // NCL
    nchunks = (max_len + CH - 1) // CH

    obuf[0, 0:NCL, :] = jnp.zeros((NCL, 2 * HIDS), jnp.float32)
    zh = pltpu.make_async_copy(obuf.at[0, pl.ds(0, NCL)],
                               h_hbm.at[pl.ds(TRASH, NCL)], soh.at[0])
    zh.start()
    zh.wait()

    def start_in(c, slot):
        pltpu.make_async_copy(pgf_hbm.at[pl.ds(c * CHR, CHR)],
                              buf.at[slot, :, pl.ds(0, GATES)],
                              sif.at[slot]).start()
        pltpu.make_async_copy(pgb_hbm.at[pl.ds(c * CHR, CHR)],
                              buf.at[slot, :, pl.ds(GATES, GATES)],
                              sib.at[slot]).start()

    @pl.when(nchunks > 0)
    def _():
        start_in(0, 0)

    def chunk_body(c, carry):
        hh, c_f, c_b = carry
        slot = lax.rem(c, 2)
        pltpu.make_async_copy(pgf_hbm.at[pl.ds(c * CHR, CHR)],
                              buf.at[slot, :, pl.ds(0, GATES)],
                              sif.at[slot]).wait()
        pltpu.make_async_copy(pgb_hbm.at[pl.ds(c * CHR, CHR)],
                              buf.at[slot, :, pl.ds(GATES, GATES)],
                              sib.at[slot]).wait()

        @pl.when(c + 1 < nchunks)
        def _():
            start_in(c + 1, lax.rem(c + 1, 2))

        @pl.when(c >= 2)
        def _():
            pltpu.make_async_copy(obuf.at[slot],
                                  h_hbm.at[pl.ds((c - 2) * CHR, CHR)],
                                  soh.at[slot]).wait()

        def step(t, sc):
            hh, c_f, c_b = sc
            base = t * NCL
            slab = buf[slot, pl.ds(base, NCL), :]
            gc = slab + jax.lax.dot_general(
                hh.astype(jnp.bfloat16), wbd_ref[...],
                (((1,), (0,)), ((), ())),
                preferred_element_type=jnp.float32,
            )
            sig_f = jax.nn.sigmoid(gc[:, 0:3 * HIDS])
            tan_f = jnp.tanh(gc[:, 3 * HIDS:4 * HIDS])
            sig_b = jax.nn.sigmoid(gc[:, GATES:GATES + 3 * HIDS])
            tan_b = jnp.tanh(gc[:, GATES + 3 * HIDS:GATES + 4 * HIDS])
            c_f = sig_f[:, HIDS:2 * HIDS] * c_f + sig_f[:, 0:HIDS] * tan_f
            c_b = sig_b[:, HIDS:2 * HIDS] * c_b + sig_b[:, 0:HIDS] * tan_b
            h_f = sig_f[:, 2 * HIDS:3 * HIDS] * jnp.tanh(c_f)
            h_b = sig_b[:, 2 * HIDS:3 * HIDS] * jnp.tanh(c_b)
            hh = jnp.concatenate([h_f, h_b], axis=1)
            obuf[slot, pl.ds(base, NCL), :] = hh
            return hh, c_f, c_b

        hh, c_f, c_b = lax.fori_loop(0, CH, step, (hh, c_f, c_b))

        pltpu.make_async_copy(obuf.at[slot],
                              h_hbm.at[pl.ds(c * CHR, CHR)],
                              soh.at[slot]).start()
        return hh, c_f, c_b

    z128 = jnp.zeros((NCL, HIDS), jnp.float32)
    z256 = jnp.zeros((NCL, 2 * HIDS), jnp.float32)
    lax.fori_loop(0, nchunks, chunk_body, (z256, z128, z128))

    @pl.when(nchunks >= 2)
    def _():
        c = nchunks - 2
        slot = lax.rem(c, 2)
        pltpu.make_async_copy(obuf.at[slot],
                              h_hbm.at[pl.ds(c * CHR, CHR)], soh.at[slot]).wait()

    @pl.when(nchunks >= 1)
    def _():
        c = nchunks - 1
        slot = lax.rem(c, 2)
        pltpu.make_async_copy(obuf.at[slot],
                              h_hbm.at[pl.ds(c * CHR, CHR)], soh.at[slot]).wait()


def _run_recurrence(d_f2d, wbd, pg_f, pg_b):
    return pl.pallas_call(
        _rec_kernel,
        in_specs=[
            pl.BlockSpec(memory_space=pltpu.MemorySpace.VMEM),
            pl.BlockSpec(memory_space=pltpu.MemorySpace.VMEM),
            pl.BlockSpec(memory_space=pltpu.MemorySpace.HBM),
            pl.BlockSpec(memory_space=pltpu.MemorySpace.HBM),
        ],
        out_specs=pl.BlockSpec(memory_space=pltpu.MemorySpace.HBM),
        out_shape=jax.ShapeDtypeStruct((RPAD, 2 * HIDS), jnp.float32),
        scratch_shapes=[
            pltpu.VMEM((2, CHR, 2 * GATES), jnp.float32),
            pltpu.VMEM((2, CHR, 2 * HIDS), jnp.float32),
            pltpu.SemaphoreType.DMA((2,)),
            pltpu.SemaphoreType.DMA((2,)),
            pltpu.SemaphoreType.DMA((2,)),
        ],
    )(d_f2d, wbd, pg_f, pg_b)


def kernel(x, context_mask, speakers, W_ih_f, W_hh_f, b_ih_f, b_hh_f,
           W_ih_b, W_hh_b, b_ih_b, b_hh_b):
    B, S, d = x.shape
    x_flat = x.reshape(MTOK, d)
    mask_i = context_mask.reshape(MTOK).astype(jnp.int32)
    spk = speakers.reshape(MTOK).astype(jnp.int32)

    def _reord(w):
        return jnp.concatenate(
            [w[:, :2 * HIDS], w[:, 3 * HIDS:], w[:, 2 * HIDS:3 * HIDS]], axis=1)

    wcat = jnp.concatenate([_reord(W_ih_f.T), _reord(W_ih_b.T)], axis=1)
    bf = b_ih_f + b_hh_f
    bb = b_ih_b + b_hh_b
    bcat = jnp.concatenate(
        [_reord(bf.reshape(1, GATES)), _reord(bb.reshape(1, GATES))], axis=1)
    gates_f, gates_b = _compute_gates(x_flat, wcat, bcat)

    dest_f, dest_b, pg_f, pg_b = _sc_group_scatter(spk, mask_i, gates_f, gates_b)

    wbd = jnp.zeros((2 * HIDS, 2 * GATES), jnp.float32)
    wbd = wbd.at[:HIDS, :GATES].set(_reord(W_hh_f.T))
    wbd = wbd.at[HIDS:, GATES:].set(_reord(W_hh_b.T))
    wbd = wbd.astype(jnp.bfloat16)

    d_f2d = dest_f.reshape(32, 128)
    h_buf = _run_recurrence(d_f2d, wbd, pg_f, pg_b)

    out = _sc_gather(h_buf, dest_f, dest_b)
    return out.reshape(B, S, 2 * HIDS)


# speedup vs baseline: 276.1724x; 3.4857x over previous
import functools

import jax
import jax.numpy as jnp
from jax import lax
from jax.experimental import pallas as pl
from jax.experimental.pallas import tpu as pltpu
from jax.experimental.pallas import tpu_sc as plsc

DIMS = 256
HIDS = 128
GATES = 4 * HIDS
NCL = 16
MTOK = 8 * 512
TRASH = MTOK * NCL
RPAD = MTOK * NCL + MTOK
CH = 64
CHR = CH * NCL


def _gates_kernel(x_ref, w_ref, b_ref, of_ref, ob_ref):
    g = (
        jax.lax.dot_general(
            x_ref[...], w_ref[...], (((1,), (0,)), ((), ())),
            preferred_element_type=jnp.float32,
            precision=jax.lax.Precision.HIGHEST,
        )
        + b_ref[...]
    )
    of_ref[...] = g[:, :GATES]
    ob_ref[...] = g[:, GATES:]


def _compute_gates(x_flat, wcat, bcat):
    return pl.pallas_call(
        _gates_kernel,
        grid=(8,),
        in_specs=[
            pl.BlockSpec((512, DIMS), lambda i: (i, 0)),
            pl.BlockSpec((DIMS, 2 * GATES), lambda i: (0, 0)),
            pl.BlockSpec((1, 2 * GATES), lambda i: (0, 0)),
        ],
        out_specs=[
            pl.BlockSpec((512, GATES), lambda i: (i, 0)),
            pl.BlockSpec((512, GATES), lambda i: (i, 0)),
        ],
        out_shape=[
            jax.ShapeDtypeStruct((MTOK, GATES), jnp.float32),
            jax.ShapeDtypeStruct((MTOK, GATES), jnp.float32),
        ],
    )(x_flat, wcat, bcat)


NSUB = 16
TPW = MTOK // 32
NCHK = MTOK // NSUB


def _sc_group_scatter_body(spk_hbm, mask_hbm, gf_hbm, gb_hbm,
                           destf_hbm, destb_hbm, pgf_hbm, pgb_hbm,
                           spk_v, mask_v, col_v, df_v, db_v, tmp_v,
                           outf_v, outb_v, rows_v, sh_f, sh_b, sem):
    c = lax.axis_index("c")
    s = lax.axis_index("s")
    k = s

    pltpu.sync_copy(spk_hbm, spk_v)
    pltpu.sync_copy(mask_hbm, mask_v)

    def p1(i, counter):
        sv = spk_v[pl.ds(i * NSUB, NSUB)]
        mv = mask_v[pl.ds(i * NSUB, NSUB)]
        m = (sv == k) & (mv != 0)
        mi = jnp.where(m, jnp.int32(1), jnp.int32(0))
        pc = plsc.cumsum(mi)
        col_v[pl.ds(i * NSUB, NSUB)] = jnp.where(m, counter + pc - 1, -1)
        return counter + jnp.sum(mi)

    len_k = lax.fori_loop(0, NCHK, p1, jnp.int32(0))

    def p2(i, carry):
        cv = col_v[pl.ds(i * NSUB, NSUB)]
        m = cv >= 0
        df_v[pl.ds(i * NSUB, NSUB)] = jnp.where(m, cv * NCL + k + 1, 0)
        db_v[pl.ds(i * NSUB, NSUB)] = jnp.where(
            m, (len_k - 1 - cv) * NCL + k + 1, 0)
        return carry

    lax.fori_loop(0, NCHK, p2, jnp.int32(0))

    pltpu.sync_copy(df_v, sh_f.at[s])
    pltpu.sync_copy(db_v, sh_b.at[s])
    plsc.subcore_barrier()

    wid = c * NSUB + s
    base = wid * TPW

    iota16 = jax.lax.iota(jnp.int32, NSUB)

    def merge(sh, out_v):
        pltpu.sync_copy(sh.at[:, pl.ds(base, TPW)], tmp_v)
        for j in range(TPW // NSUB):
            acc = jnp.zeros((NSUB,), jnp.int32)
            for r in range(NSUB):
                acc = acc + tmp_v[r, pl.ds(j * NSUB, NSUB)]
            trash = TRASH + base + j * NSUB + iota16
            out_v[pl.ds(j * NSUB, NSUB)] = jnp.where(acc > 0, acc - 1, trash)

    merge(sh_f, outf_v)
    merge(sh_b, outb_v)
    pltpu.sync_copy(outf_v, destf_hbm.at[pl.ds(base, TPW)])
    pltpu.sync_copy(outb_v, destb_hbm.at[pl.ds(base, TPW)])

    pltpu.sync_copy(gf_hbm.at[pl.ds(base, TPW)], rows_v)
    pltpu.async_copy(rows_v, pgf_hbm.at[outf_v], sem).wait()
    pltpu.sync_copy(gb_hbm.at[pl.ds(base, TPW)], rows_v)
    pltpu.async_copy(rows_v, pgb_hbm.at[outb_v], sem).wait()


def _sc_group_scatter(spk, mask_i, gates_f, gates_b):
    mesh = plsc.VectorSubcoreMesh(core_axis_name="c", subcore_axis_name="s")
    f = pl.kernel(
        _sc_group_scatter_body,
        out_type=[
            jax.ShapeDtypeStruct((MTOK,), jnp.int32),
            jax.ShapeDtypeStruct((MTOK,), jnp.int32),
            jax.ShapeDtypeStruct((RPAD, GATES), jnp.float32),
            jax.ShapeDtypeStruct((RPAD, GATES), jnp.float32),
        ],
        mesh=mesh,
        scratch_types=[
            pltpu.VMEM((MTOK,), jnp.int32),
            pltpu.VMEM((MTOK,), jnp.int32),
            pltpu.VMEM((MTOK,), jnp.int32),
            pltpu.VMEM((MTOK,), jnp.int32),
            pltpu.VMEM((MTOK,), jnp.int32),
            pltpu.VMEM((NSUB, TPW), jnp.int32),
            pltpu.VMEM((TPW,), jnp.int32),
            pltpu.VMEM((TPW,), jnp.int32),
            pltpu.VMEM((TPW, GATES), jnp.float32),
            pltpu.MemorySpace.VMEM_SHARED((NSUB, MTOK), jnp.int32),
            pltpu.MemorySpace.VMEM_SHARED((NSUB, MTOK), jnp.int32),
            pltpu.SemaphoreType.DMA,
        ],
        compiler_params=pltpu.CompilerParams(needs_layout_passes=False),
    )
    return f(spk, mask_i, gates_f, gates_b)


def _sc_gather_body(h_hbm, destf_hbm, destb_hbm, out_hbm,
                    idx_v, rowsf_v, rowsb_v, sem):
    c = lax.axis_index("c")
    s = lax.axis_index("s")
    wid = c * NSUB + s
    base = wid * TPW
    pltpu.sync_copy(destf_hbm.at[pl.ds(base, TPW)], idx_v)
    pltpu.async_copy(h_hbm.at[idx_v], rowsf_v, sem).wait()
    pltpu.sync_copy(destb_hbm.at[pl.ds(base, TPW)], idx_v)
    pltpu.async_copy(h_hbm.at[idx_v], rowsb_v, sem).wait()
    pltpu.sync_copy(rowsf_v.at[:, pl.ds(0, HIDS)],
                    out_hbm.at[pl.ds(base, TPW), pl.ds(0, HIDS)])
    pltpu.sync_copy(rowsb_v.at[:, pl.ds(HIDS, HIDS)],
                    out_hbm.at[pl.ds(base, TPW), pl.ds(HIDS, HIDS)])


def _sc_gather(h_buf, dest_f, dest_b):
    mesh = plsc.VectorSubcoreMesh(core_axis_name="c", subcore_axis_name="s")
    f = pl.kernel(
        _sc_gather_body,
        out_type=jax.ShapeDtypeStruct((MTOK, 2 * HIDS), jnp.float32),
        mesh=mesh,
        scratch_types=[
            pltpu.VMEM((TPW,), jnp.int32),
            pltpu.VMEM((TPW, 2 * HIDS), jnp.float32),
            pltpu.VMEM((TPW, 2 * HIDS), jnp.float32),
            pltpu.SemaphoreType.DMA,
        ],
        compiler_params=pltpu.CompilerParams(needs_layout_passes=False),
    )
    return f(h_buf, dest_f, dest_b)


def _rec_kernel(df_ref, wbd_ref, pgf_hbm, pgb_hbm, h_hbm,
                buf0, buf1, obuf0, obuf1, sif, sib, soh):
    dv = jnp.where(df_ref[...] >= TRASH, -1, df_ref[...])
    maxv = jnp.max(dv)
    max_len = (maxv + NCL) // NCL
    nchunks = (max_len + CH - 1) // CH
    npairs = (nchunks + 1) // 2

    obuf0[...] = jnp.zeros((CHR, 2 * HIDS), jnp.float32)
    for q in range(MTOK // CHR):
        pltpu.make_async_copy(obuf0,
                              h_hbm.at[pl.ds(TRASH + q * CHR, CHR)],
                              soh.at[0]).start()
    for q in range(MTOK // CHR):
        pltpu.make_async_copy(obuf0,
                              h_hbm.at[pl.ds(TRASH + q * CHR, CHR)],
                              soh.at[0]).wait()

    bufs = (buf0, buf1)
    obufs = (obuf0, obuf1)

    def start_in(c, slot):
        pltpu.make_async_copy(pgf_hbm.at[pl.ds(c * CHR, CHR)],
                              bufs[slot].at[:, pl.ds(0, GATES)],
                              sif.at[slot]).start()
        pltpu.make_async_copy(pgb_hbm.at[pl.ds(c * CHR, CHR)],
                              bufs[slot].at[:, pl.ds(GATES, GATES)],
                              sib.at[slot]).start()

    @pl.when(nchunks > 0)
    def _():
        start_in(0, 0)
        start_in(1, 1)

    def run_steps(slot, carry):
        hh, c_f, c_b = carry
        bufx = bufs[slot]
        obufx = obufs[slot]
        for u in range(CH):
            base = u * NCL
            slab = bufx[pl.ds(base, NCL), :]
            gc = slab + jax.lax.dot_general(
                hh.astype(jnp.bfloat16), wbd_ref[...],
                (((1,), (0,)), ((), ())),
                preferred_element_type=jnp.float32,
            )
            sig_f = jax.nn.sigmoid(gc[:, 0:3 * HIDS])
            tan_f = jnp.tanh(gc[:, 3 * HIDS:4 * HIDS])
            sig_b = jax.nn.sigmoid(gc[:, GATES:GATES + 3 * HIDS])
            tan_b = jnp.tanh(gc[:, GATES + 3 * HIDS:GATES + 4 * HIDS])
            c_f = sig_f[:, HIDS:2 * HIDS] * c_f + sig_f[:, 0:HIDS] * tan_f
            c_b = sig_b[:, HIDS:2 * HIDS] * c_b + sig_b[:, 0:HIDS] * tan_b
            h_f = sig_f[:, 2 * HIDS:3 * HIDS] * jnp.tanh(c_f)
            h_b = sig_b[:, 2 * HIDS:3 * HIDS] * jnp.tanh(c_b)
            hh = jnp.concatenate([h_f, h_b], axis=1)
            obufx[pl.ds(base, NCL), :] = hh
        return hh, c_f, c_b

    def pair_body(p, carry):
        c0 = 2 * p
        for slot in (0, 1):
            c = c0 + slot
            pltpu.make_async_copy(pgf_hbm.at[pl.ds(c * CHR, CHR)],
                                  bufs[slot].at[:, pl.ds(0, GATES)],
                                  sif.at[slot]).wait()
            pltpu.make_async_copy(pgb_hbm.at[pl.ds(c * CHR, CHR)],
                                  bufs[slot].at[:, pl.ds(GATES, GATES)],
                                  sib.at[slot]).wait()

            @pl.when(p >= 1)
            def _():
                pltpu.make_async_copy(obufs[slot],
                                      h_hbm.at[pl.ds((c - 2) * CHR, CHR)],
                                      soh.at[slot]).wait()

            carry = run_steps(slot, carry)

            @pl.when(c + 2 < 2 * npairs)
            def _():
                start_in(c + 2, slot)

            pltpu.make_async_copy(obufs[slot],
                                  h_hbm.at[pl.ds(c * CHR, CHR)],
                                  soh.at[slot]).start()
        return carry

    z128 = jnp.zeros((NCL, HIDS), jnp.float32)
    z256 = jnp.zeros((NCL, 2 * HIDS), jnp.float32)
    lax.fori_loop(0, npairs, pair_body, (z256, z128, z128))

    @pl.when(npairs >= 1)
    def _():
        pltpu.make_async_copy(obuf0,
                              h_hbm.at[pl.ds((2 * npairs - 2) * CHR, CHR)],
                              soh.at[0]).wait()
        pltpu.make_async_copy(obuf1,
                              h_hbm.at[pl.ds((2 * npairs - 1) * CHR, CHR)],
                              soh.at[1]).wait()


def _run_recurrence(d_f2d, wbd, pg_f, pg_b):
    return pl.pallas_call(
        _rec_kernel,
        in_specs=[
            pl.BlockSpec(memory_space=pltpu.MemorySpace.VMEM),
            pl.BlockSpec(memory_space=pltpu.MemorySpace.VMEM),
            pl.BlockSpec(memory_space=pltpu.MemorySpace.HBM),
            pl.BlockSpec(memory_space=pltpu.MemorySpace.HBM),
        ],
        out_specs=pl.BlockSpec(memory_space=pltpu.MemorySpace.HBM),
        out_shape=jax.ShapeDtypeStruct((RPAD, 2 * HIDS), jnp.float32),
        scratch_shapes=[
            pltpu.VMEM((CHR, 2 * GATES), jnp.float32),
            pltpu.VMEM((CHR, 2 * GATES), jnp.float32),
            pltpu.VMEM((CHR, 2 * HIDS), jnp.float32),
            pltpu.VMEM((CHR, 2 * HIDS), jnp.float32),
            pltpu.SemaphoreType.DMA((2,)),
            pltpu.SemaphoreType.DMA((2,)),
            pltpu.SemaphoreType.DMA((2,)),
        ],
    )(d_f2d, wbd, pg_f, pg_b)


def kernel(x, context_mask, speakers, W_ih_f, W_hh_f, b_ih_f, b_hh_f,
           W_ih_b, W_hh_b, b_ih_b, b_hh_b):
    B, S, d = x.shape
    x_flat = x.reshape(MTOK, d)
    mask_i = context_mask.reshape(MTOK).astype(jnp.int32)
    spk = speakers.reshape(MTOK).astype(jnp.int32)

    def _reord(w):
        return jnp.concatenate(
            [w[:, :2 * HIDS], w[:, 3 * HIDS:], w[:, 2 * HIDS:3 * HIDS]], axis=1)

    wcat = jnp.concatenate([_reord(W_ih_f.T), _reord(W_ih_b.T)], axis=1)
    bf = b_ih_f + b_hh_f
    bb = b_ih_b + b_hh_b
    bcat = jnp.concatenate(
        [_reord(bf.reshape(1, GATES)), _reord(bb.reshape(1, GATES))], axis=1)
    gates_f, gates_b = _compute_gates(x_flat, wcat, bcat)

    dest_f, dest_b, pg_f, pg_b = _sc_group_scatter(spk, mask_i, gates_f, gates_b)

    wbd = jnp.zeros((2 * HIDS, 2 * GATES), jnp.float32)
    wbd = wbd.at[:HIDS, :GATES].set(_reord(W_hh_f.T))
    wbd = wbd.at[HIDS:, GATES:].set(_reord(W_hh_b.T))
    wbd = wbd.astype(jnp.bfloat16)

    d_f2d = dest_f.reshape(32, 128)
    h_buf = _run_recurrence(d_f2d, wbd, pg_f, pg_b)

    out = _sc_gather(h_buf, dest_f, dest_b)
    return out.reshape(B, S, 2 * HIDS)


# speedup vs baseline: 328.2421x; 1.1885x over previous
import functools

import jax
import jax.numpy as jnp
from jax import lax
from jax.experimental import pallas as pl
from jax.experimental.pallas import tpu as pltpu
from jax.experimental.pallas import tpu_sc as plsc

DIMS = 256
HIDS = 128
GATES = 4 * HIDS
NCL = 16
MTOK = 8 * 512
TRASH = MTOK * NCL
RPAD = MTOK * NCL + MTOK
CH = 32
CHR = CH * NCL


def _gates_kernel(x_ref, w_ref, b_ref, of_ref, ob_ref):
    g = (
        jax.lax.dot_general(
            x_ref[...].astype(jnp.bfloat16), w_ref[...],
            (((1,), (0,)), ((), ())),
            preferred_element_type=jnp.float32,
        )
        + b_ref[...]
    )
    of_ref[...] = g[:, :GATES]
    ob_ref[...] = g[:, GATES:]


def _compute_gates(x_flat, wcat, bcat):
    return pl.pallas_call(
        _gates_kernel,
        grid=(8,),
        in_specs=[
            pl.BlockSpec((512, DIMS), lambda i: (i, 0)),
            pl.BlockSpec((DIMS, 2 * GATES), lambda i: (0, 0)),
            pl.BlockSpec((1, 2 * GATES), lambda i: (0, 0)),
        ],
        out_specs=[
            pl.BlockSpec((512, GATES), lambda i: (i, 0)),
            pl.BlockSpec((512, GATES), lambda i: (i, 0)),
        ],
        out_shape=[
            jax.ShapeDtypeStruct((MTOK, GATES), jnp.float32),
            jax.ShapeDtypeStruct((MTOK, GATES), jnp.float32),
        ],
    )(x_flat, wcat, bcat)


NSUB = 16
TPW = MTOK // 32
NCHK = MTOK // NSUB


def _sc_group_scatter_body(spk_hbm, mask_hbm, gf_hbm, gb_hbm,
                           destf_hbm, destb_hbm, pgf_hbm, pgb_hbm,
                           spk_v, mask_v, col_v, df_v, db_v, tmp_v,
                           outf_v, outb_v, rows_v, sh_f, sh_b, sem):
    c = lax.axis_index("c")
    s = lax.axis_index("s")
    k = s

    pltpu.sync_copy(spk_hbm, spk_v)
    pltpu.sync_copy(mask_hbm, mask_v)

    def p1(i, counter):
        sv = spk_v[pl.ds(i * NSUB, NSUB)]
        mv = mask_v[pl.ds(i * NSUB, NSUB)]
        m = (sv == k) & (mv != 0)
        mi = jnp.where(m, jnp.int32(1), jnp.int32(0))
        pc = plsc.cumsum(mi)
        col_v[pl.ds(i * NSUB, NSUB)] = jnp.where(m, counter + pc - 1, -1)
        return counter + jnp.sum(mi)

    len_k = lax.fori_loop(0, NCHK, p1, jnp.int32(0))

    def p2(i, carry):
        cv = col_v[pl.ds(i * NSUB, NSUB)]
        m = cv >= 0
        df_v[pl.ds(i * NSUB, NSUB)] = jnp.where(m, cv * NCL + k + 1, 0)
        db_v[pl.ds(i * NSUB, NSUB)] = jnp.where(
            m, (len_k - 1 - cv) * NCL + k + 1, 0)
        return carry

    lax.fori_loop(0, NCHK, p2, jnp.int32(0))

    pltpu.sync_copy(df_v, sh_f.at[s])
    pltpu.sync_copy(db_v, sh_b.at[s])
    plsc.subcore_barrier()

    wid = c * NSUB + s
    base = wid * TPW

    iota16 = jax.lax.iota(jnp.int32, NSUB)

    def merge(sh, out_v):
        pltpu.sync_copy(sh.at[:, pl.ds(base, TPW)], tmp_v)
        for j in range(TPW // NSUB):
            acc = jnp.zeros((NSUB,), jnp.int32)
            for r in range(NSUB):
                acc = acc + tmp_v[r, pl.ds(j * NSUB, NSUB)]
            trash = TRASH + base + j * NSUB + iota16
            out_v[pl.ds(j * NSUB, NSUB)] = jnp.where(acc > 0, acc - 1, trash)

    merge(sh_f, outf_v)
    merge(sh_b, outb_v)
    pltpu.sync_copy(outf_v, destf_hbm.at[pl.ds(base, TPW)])
    pltpu.sync_copy(outb_v, destb_hbm.at[pl.ds(base, TPW)])

    pltpu.sync_copy(gf_hbm.at[pl.ds(base, TPW)], rows_v)
    pltpu.async_copy(rows_v, pgf_hbm.at[outf_v], sem).wait()
    pltpu.sync_copy(gb_hbm.at[pl.ds(base, TPW)], rows_v)
    pltpu.async_copy(rows_v, pgb_hbm.at[outb_v], sem).wait()


def _sc_group_scatter(spk, mask_i, gates_f, gates_b):
    mesh = plsc.VectorSubcoreMesh(core_axis_name="c", subcore_axis_name="s")
    f = pl.kernel(
        _sc_group_scatter_body,
        out_type=[
            jax.ShapeDtypeStruct((MTOK,), jnp.int32),
            jax.ShapeDtypeStruct((MTOK,), jnp.int32),
            jax.ShapeDtypeStruct((RPAD, GATES), jnp.float32),
            jax.ShapeDtypeStruct((RPAD, GATES), jnp.float32),
        ],
        mesh=mesh,
        scratch_types=[
            pltpu.VMEM((MTOK,), jnp.int32),
            pltpu.VMEM((MTOK,), jnp.int32),
            pltpu.VMEM((MTOK,), jnp.int32),
            pltpu.VMEM((MTOK,), jnp.int32),
            pltpu.VMEM((MTOK,), jnp.int32),
            pltpu.VMEM((NSUB, TPW), jnp.int32),
            pltpu.VMEM((TPW,), jnp.int32),
            pltpu.VMEM((TPW,), jnp.int32),
            pltpu.VMEM((TPW, GATES), jnp.float32),
            pltpu.MemorySpace.VMEM_SHARED((NSUB, MTOK), jnp.int32),
            pltpu.MemorySpace.VMEM_SHARED((NSUB, MTOK), jnp.int32),
            pltpu.SemaphoreType.DMA,
        ],
        compiler_params=pltpu.CompilerParams(needs_layout_passes=False),
    )
    return f(spk, mask_i, gates_f, gates_b)


def _sc_gather_body(h_hbm, destf_hbm, destb_hbm, out_hbm,
                    idx_v, rowsf_v, rowsb_v, sem):
    c = lax.axis_index("c")
    s = lax.axis_index("s")
    wid = c * NSUB + s
    base = wid * TPW
    pltpu.sync_copy(destf_hbm.at[pl.ds(base, TPW)], idx_v)
    pltpu.async_copy(h_hbm.at[idx_v], rowsf_v, sem).wait()
    pltpu.sync_copy(destb_hbm.at[pl.ds(base, TPW)], idx_v)
    pltpu.async_copy(h_hbm.at[idx_v], rowsb_v, sem).wait()
    pltpu.sync_copy(rowsf_v.at[:, pl.ds(0, HIDS)],
                    out_hbm.at[pl.ds(base, TPW), pl.ds(0, HIDS)])
    pltpu.sync_copy(rowsb_v.at[:, pl.ds(HIDS, HIDS)],
                    out_hbm.at[pl.ds(base, TPW), pl.ds(HIDS, HIDS)])


def _sc_gather(h_buf, dest_f, dest_b):
    mesh = plsc.VectorSubcoreMesh(core_axis_name="c", subcore_axis_name="s")
    f = pl.kernel(
        _sc_gather_body,
        out_type=jax.ShapeDtypeStruct((MTOK, 2 * HIDS), jnp.float32),
        mesh=mesh,
        scratch_types=[
            pltpu.VMEM((TPW,), jnp.int32),
            pltpu.VMEM((TPW, 2 * HIDS), jnp.float32),
            pltpu.VMEM((TPW, 2 * HIDS), jnp.float32),
            pltpu.SemaphoreType.DMA,
        ],
        compiler_params=pltpu.CompilerParams(needs_layout_passes=False),
    )
    return f(h_buf, dest_f, dest_b)


def _rec_kernel(df_ref, wbd_ref, pgf_hbm, pgb_hbm, h_hbm,
                buf0, buf1, obuf0, obuf1, sif, sib, soh):
    dv = jnp.where(df_ref[...] >= TRASH, -1, df_ref[...])
    maxv = jnp.max(dv)
    max_len = (maxv + NCL) // NCL
    nchunks = (max_len + CH - 1) // CH
    npairs = (nchunks + 1) // 2

    obuf0[...] = jnp.zeros((CHR, 2 * HIDS), jnp.float32)
    for q in range(MTOK // CHR):
        pltpu.make_async_copy(obuf0,
                              h_hbm.at[pl.ds(TRASH + q * CHR, CHR)],
                              soh.at[0]).start()
    for q in range(MTOK // CHR):
        pltpu.make_async_copy(obuf0,
                              h_hbm.at[pl.ds(TRASH + q * CHR, CHR)],
                              soh.at[0]).wait()

    bufs = (buf0, buf1)
    obufs = (obuf0, obuf1)

    def start_in(c, slot):
        pltpu.make_async_copy(pgf_hbm.at[pl.ds(c * CHR, CHR)],
                              bufs[slot].at[:, pl.ds(0, GATES)],
                              sif.at[slot]).start()
        pltpu.make_async_copy(pgb_hbm.at[pl.ds(c * CHR, CHR)],
                              bufs[slot].at[:, pl.ds(GATES, GATES)],
                              sib.at[slot]).start()

    @pl.when(nchunks > 0)
    def _():
        start_in(0, 0)
        start_in(1, 1)

    def run_steps(slot, carry):
        hh, c_f, c_b = carry
        bufx = bufs[slot]
        obufx = obufs[slot]
        for u in range(CH):
            base = u * NCL
            slab = bufx[pl.ds(base, NCL), :]
            gc = slab + jax.lax.dot_general(
                hh.astype(jnp.bfloat16), wbd_ref[...],
                (((1,), (0,)), ((), ())),
                preferred_element_type=jnp.float32,
            )
            sig_f = jax.nn.sigmoid(gc[:, 0:3 * HIDS])
            tan_f = jnp.tanh(gc[:, 3 * HIDS:4 * HIDS])
            sig_b = jax.nn.sigmoid(gc[:, GATES:GATES + 3 * HIDS])
            tan_b = jnp.tanh(gc[:, GATES + 3 * HIDS:GATES + 4 * HIDS])
            c_f = sig_f[:, HIDS:2 * HIDS] * c_f + sig_f[:, 0:HIDS] * tan_f
            c_b = sig_b[:, HIDS:2 * HIDS] * c_b + sig_b[:, 0:HIDS] * tan_b
            h_f = sig_f[:, 2 * HIDS:3 * HIDS] * jnp.tanh(c_f)
            h_b = sig_b[:, 2 * HIDS:3 * HIDS] * jnp.tanh(c_b)
            hh = jnp.concatenate([h_f, h_b], axis=1)
            obufx[pl.ds(base, NCL), :] = hh
        return hh, c_f, c_b

    def pair_body(p, carry):
        c0 = 2 * p
        for slot in (0, 1):
            c = c0 + slot
            pltpu.make_async_copy(pgf_hbm.at[pl.ds(c * CHR, CHR)],
                                  bufs[slot].at[:, pl.ds(0, GATES)],
                                  sif.at[slot]).wait()
            pltpu.make_async_copy(pgb_hbm.at[pl.ds(c * CHR, CHR)],
                                  bufs[slot].at[:, pl.ds(GATES, GATES)],
                                  sib.at[slot]).wait()

            @pl.when(p >= 1)
            def _():
                pltpu.make_async_copy(obufs[slot],
                                      h_hbm.at[pl.ds((c - 2) * CHR, CHR)],
                                      soh.at[slot]).wait()

            carry = run_steps(slot, carry)

            @pl.when(c + 2 < 2 * npairs)
            def _():
                start_in(c + 2, slot)

            pltpu.make_async_copy(obufs[slot],
                                  h_hbm.at[pl.ds(c * CHR, CHR)],
                                  soh.at[slot]).start()
        return carry

    z128 = jnp.zeros((NCL, HIDS), jnp.float32)
    z256 = jnp.zeros((NCL, 2 * HIDS), jnp.float32)
    lax.fori_loop(0, npairs, pair_body, (z256, z128, z128))

    @pl.when(npairs >= 1)
    def _():
        pltpu.make_async_copy(obuf0,
                              h_hbm.at[pl.ds((2 * npairs - 2) * CHR, CHR)],
                              soh.at[0]).wait()
        pltpu.make_async_copy(obuf1,
                              h_hbm.at[pl.ds((2 * npairs - 1) * CHR, CHR)],
                              soh.at[1]).wait()


def _run_recurrence(d_f2d, wbd, pg_f, pg_b):
    return pl.pallas_call(
        _rec_kernel,
        in_specs=[
            pl.BlockSpec(memory_space=pltpu.MemorySpace.VMEM),
            pl.BlockSpec(memory_space=pltpu.MemorySpace.VMEM),
            pl.BlockSpec(memory_space=pltpu.MemorySpace.HBM),
            pl.BlockSpec(memory_space=pltpu.MemorySpace.HBM),
        ],
        out_specs=pl.BlockSpec(memory_space=pltpu.MemorySpace.HBM),
        out_shape=jax.ShapeDtypeStruct((RPAD, 2 * HIDS), jnp.float32),
        scratch_shapes=[
            pltpu.VMEM((CHR, 2 * GATES), jnp.float32),
            pltpu.VMEM((CHR, 2 * GATES), jnp.float32),
            pltpu.VMEM((CHR, 2 * HIDS), jnp.float32),
            pltpu.VMEM((CHR, 2 * HIDS), jnp.float32),
            pltpu.SemaphoreType.DMA((2,)),
            pltpu.SemaphoreType.DMA((2,)),
            pltpu.SemaphoreType.DMA((2,)),
        ],
    )(d_f2d, wbd, pg_f, pg_b)


def kernel(x, context_mask, speakers, W_ih_f, W_hh_f, b_ih_f, b_hh_f,
           W_ih_b, W_hh_b, b_ih_b, b_hh_b):
    B, S, d = x.shape
    x_flat = x.reshape(MTOK, d)
    mask_i = context_mask.reshape(MTOK).astype(jnp.int32)
    spk = speakers.reshape(MTOK).astype(jnp.int32)

    def _reord(w):
        return jnp.concatenate(
            [w[:, :2 * HIDS], w[:, 3 * HIDS:], w[:, 2 * HIDS:3 * HIDS]], axis=1)

    wcat = jnp.concatenate([_reord(W_ih_f.T), _reord(W_ih_b.T)], axis=1)
    bf = b_ih_f + b_hh_f
    bb = b_ih_b + b_hh_b
    bcat = jnp.concatenate(
        [_reord(bf.reshape(1, GATES)), _reord(bb.reshape(1, GATES))], axis=1)
    gates_f, gates_b = _compute_gates(x_flat, wcat.astype(jnp.bfloat16), bcat)

    dest_f, dest_b, pg_f, pg_b = _sc_group_scatter(spk, mask_i, gates_f, gates_b)

    wbd = jnp.zeros((2 * HIDS, 2 * GATES), jnp.float32)
    wbd = wbd.at[:HIDS, :GATES].set(_reord(W_hh_f.T))
    wbd = wbd.at[HIDS:, GATES:].set(_reord(W_hh_b.T))
    wbd = wbd.astype(jnp.bfloat16)

    d_f2d = dest_f.reshape(32, 128)
    h_buf = _run_recurrence(d_f2d, wbd, pg_f, pg_b)

    out = _sc_gather(h_buf, dest_f, dest_b)
    return out.reshape(B, S, 2 * HIDS)
